# Initial kernel scaffold; baseline (speedup 1.0000x reference)
import jax
import jax.numpy as jnp
from jax.experimental import pallas as pl


def kernel(x, edge_index, W1l, b1, W1r, W2l, b2, W2r):
    raise NotImplementedError("write your pallas kernel here")


# speedup vs baseline: 8.0240x; 8.0240x over previous
import functools

import jax
import jax.numpy as jnp
from jax import lax
from jax.experimental import pallas as pl
from jax.experimental.pallas import tpu as pltpu
from jax.experimental.pallas import tpu_sc as plsc

_NC = 2
_NS = 16
_NW = _NC * _NS
_K = 128

_F32 = jnp.float32
_HIGH = jax.lax.Precision.HIGHEST


def _tc_matmul(a, w):
    m, _ = a.shape
    n = w.shape[1]

    def body(a_ref, w_ref, o_ref):
        o_ref[...] = jnp.dot(a_ref[...], w_ref[...],
                             preferred_element_type=_F32, precision=_HIGH)

    return pl.pallas_call(
        body, out_shape=jax.ShapeDtypeStruct((m, n), _F32))(a, w)


def _tc_layer1_post(aggp, degp, r1, b1, w2l):
    n, d = r1.shape
    dout = w2l.shape[1]

    def body(agg_ref, deg_ref, r1_ref, b1_ref, w2l_ref, h_ref, y2_ref):
        deg = deg_ref[0][:n, 0:1] + deg_ref[1][:n, 0:1]
        mean = (agg_ref[0][:n] + agg_ref[1][:n]) / jnp.maximum(deg, 1.0)
        h = jnp.maximum(mean + b1_ref[...] + r1_ref[...], 0.0)
        h_ref[...] = h
        y2_ref[...] = jnp.dot(h, w2l_ref[...],
                              preferred_element_type=_F32, precision=_HIGH)

    return pl.pallas_call(
        body,
        out_shape=[jax.ShapeDtypeStruct((n, d), _F32),
                   jax.ShapeDtypeStruct((n, dout), _F32)],
    )(aggp, degp, r1, b1.reshape(1, -1), w2l)


def _tc_layer2_post(aggp, degp, r2, b2):
    n, d = r2.shape

    def body(agg_ref, deg_ref, r2_ref, b2_ref, z_ref):
        deg = deg_ref[0][:n, 0:1] + deg_ref[1][:n, 0:1]
        mean = (agg_ref[0][:n] + agg_ref[1][:n]) / jnp.maximum(deg, 1.0)
        z_ref[...] = mean + b2_ref[...] + r2_ref[...]

    return pl.pallas_call(
        body, out_shape=jax.ShapeDtypeStruct((n, d), _F32),
    )(aggp, degp, r2, b2.reshape(1, -1))


def _sc_segment_sum(y, src2d, dst2d, with_deg):
    n, d = y.shape
    rows = src2d.shape[0]
    niter = (rows + _NW - 1) // _NW
    npad = ((n + 127) // 128) * 128
    rps = npad // _NS

    mesh = plsc.VectorSubcoreMesh(core_axis_name="c", subcore_axis_name="s")
    out_type = [jax.ShapeDtypeStruct((_NC, npad, d), _F32)]
    scratch = [
        pltpu.VMEM_SHARED((npad, d), _F32),
        pltpu.VMEM((rps, d), _F32),
        pltpu.VMEM((_K,), jnp.int32),
        pltpu.VMEM((_K,), jnp.int32),
        pltpu.VMEM((_K, d), _F32),
        pltpu.SemaphoreType.DMA,
    ]
    if with_deg:
        out_type.append(jax.ShapeDtypeStruct((_NC, npad, 16), _F32))
        scratch += [
            pltpu.VMEM_SHARED((npad, 16), _F32),
            pltpu.VMEM((rps, 16), _F32),
            pltpu.VMEM((_K, 16), _F32),
        ]

    def body(y_hbm, src_hbm, dst_hbm, out_hbm, *rest):
        if with_deg:
            (deg_hbm, acc, zbuf, isrc, idst, grows, sem,
             dacc, dzbuf, ones) = rest
        else:
            acc, zbuf, isrc, idst, grows, sem = rest
        c = lax.axis_index("c")
        s = lax.axis_index("s")
        wid = s * _NC + c
        zero16 = jnp.zeros((16,), _F32)

        @pl.loop(0, rps)
        def _(i):
            @pl.loop(0, d, step=16)
            def _(j):
                zbuf[i, pl.ds(j, 16)] = zero16

        pltpu.sync_copy(zbuf, acc.at[pl.ds(s * rps, rps)])
        if with_deg:
            one16 = jnp.ones((16,), _F32)

            @pl.loop(0, rps)
            def _(i):
                dzbuf[i] = zero16

            @pl.loop(0, _K)
            def _(i):
                ones[i] = one16

            pltpu.sync_copy(dzbuf, dacc.at[pl.ds(s * rps, rps)])
        plsc.subcore_barrier()

        @pl.loop(0, niter)
        def _(i):
            r = i * _NW + wid

            @pl.when(r < rows)
            def _():
                pltpu.sync_copy(src_hbm.at[r], isrc)
                pltpu.sync_copy(dst_hbm.at[r], idst)
                pltpu.async_copy(y_hbm.at[isrc], grows, sem).wait()
                pltpu.sync_copy(grows, acc.at[idst], add=True)
                if with_deg:
                    pltpu.sync_copy(ones, dacc.at[idst], add=True)

        plsc.subcore_barrier()
        pltpu.sync_copy(acc.at[pl.ds(s * rps, rps)],
                        out_hbm.at[c].at[pl.ds(s * rps, rps)])
        if with_deg:
            pltpu.sync_copy(dacc.at[pl.ds(s * rps, rps)],
                            deg_hbm.at[c].at[pl.ds(s * rps, rps)])

    f = pl.kernel(
        body, out_type=out_type, mesh=mesh, scratch_types=scratch,
        compiler_params=pltpu.CompilerParams(use_tc_tiling_on_sc=False))
    res = f(y, src2d, dst2d)
    return tuple(res) if with_deg else res[0]


def kernel(x, edge_index, W1l, b1, W1r, W2l, b2, W2r):
    e = edge_index.shape[1]
    src2d = edge_index[0].reshape(e // _K, _K)
    dst2d = edge_index[1].reshape(e // _K, _K)

    y1 = _tc_matmul(x, W1l)
    r1 = _tc_matmul(x, W1r)
    aggp, degp = _sc_segment_sum(y1, src2d, dst2d, with_deg=True)
    h, y2 = _tc_layer1_post(aggp, degp, r1, b1, W2l)
    r2 = _tc_matmul(h, W2r)
    agg2p = _sc_segment_sum(y2, src2d, dst2d, with_deg=False)
    return _tc_layer2_post(agg2p, degp, r2, b2)


# speedup vs baseline: 16.1815x; 2.0166x over previous
import functools

import jax
import jax.numpy as jnp
from jax import lax
from jax.experimental import pallas as pl
from jax.experimental.pallas import tpu as pltpu
from jax.experimental.pallas import tpu_sc as plsc

_NC = 2
_NS = 16
_NW = _NC * _NS
_K = 128

_F32 = jnp.float32
_HIGH = jax.lax.Precision.HIGHEST


def _tc_matmul(a, w):
    m, _ = a.shape
    n = w.shape[1]

    def body(a_ref, w_ref, o_ref):
        o_ref[...] = jnp.dot(a_ref[...], w_ref[...],
                             preferred_element_type=_F32, precision=_HIGH)

    return pl.pallas_call(
        body, out_shape=jax.ShapeDtypeStruct((m, n), _F32))(a, w)


def _tc_layer1_post(aggp, degp, r1, b1, w2l):
    n, d = r1.shape
    dout = w2l.shape[1]

    def body(agg_ref, deg_ref, r1_ref, b1_ref, w2l_ref, h_ref, y2_ref):
        deg = deg_ref[0][:n, 0:1] + deg_ref[1][:n, 0:1]
        mean = (agg_ref[0][:n] + agg_ref[1][:n]) / jnp.maximum(deg, 1.0)
        h = jnp.maximum(mean + b1_ref[...] + r1_ref[...], 0.0)
        h_ref[...] = h
        y2_ref[...] = jnp.dot(h, w2l_ref[...],
                              preferred_element_type=_F32, precision=_HIGH)

    return pl.pallas_call(
        body,
        out_shape=[jax.ShapeDtypeStruct((n, d), _F32),
                   jax.ShapeDtypeStruct((n, dout), _F32)],
    )(aggp, degp, r1, b1.reshape(1, -1), w2l)


def _tc_layer2_post(aggp, degp, r2, b2):
    n, d = r2.shape

    def body(agg_ref, deg_ref, r2_ref, b2_ref, z_ref):
        deg = deg_ref[0][:n, 0:1] + deg_ref[1][:n, 0:1]
        mean = (agg_ref[0][:n] + agg_ref[1][:n]) / jnp.maximum(deg, 1.0)
        z_ref[...] = mean + b2_ref[...] + r2_ref[...]

    return pl.pallas_call(
        body, out_shape=jax.ShapeDtypeStruct((n, d), _F32),
    )(aggp, degp, r2, b2.reshape(1, -1))


_BS = 256


def _sc_segment_sum(y, srcb, dstb, src2d, dst2d, with_deg):
    n, d = y.shape
    nblk = srcb.shape[0] // _NW
    tail = src2d.shape[0] - (srcb.shape[0] * _BS) // _K
    trow0 = (srcb.shape[0] * _BS) // _K
    npad = ((n + 127) // 128) * 128
    rps = npad // _NS
    dzr = rps // 4
    assert rps % 4 == 0

    mesh = plsc.VectorSubcoreMesh(core_axis_name="c", subcore_axis_name="s")
    out_type = [jax.ShapeDtypeStruct((_NC, npad, d), _F32)]
    scratch = [
        pltpu.VMEM_SHARED((npad, d), _F32),
        pltpu.VMEM((nblk, _BS), jnp.int32),
        pltpu.VMEM((nblk, _BS), jnp.int32),
        pltpu.VMEM((_BS, d), _F32),
        pltpu.VMEM((_BS, d), _F32),
        pltpu.VMEM((_K,), jnp.int32),
        pltpu.VMEM((_K,), jnp.int32),
        pltpu.SemaphoreType.DMA,
        pltpu.SemaphoreType.DMA,
    ]
    if with_deg:
        out_type.append(jax.ShapeDtypeStruct((_NC, npad, 16), _F32))
        scratch += [
            pltpu.VMEM_SHARED((npad, 16), _F32),
            pltpu.VMEM((dzr, 16), _F32),
            pltpu.VMEM((_BS, 16), _F32),
        ]

    def body(y_hbm, srcb_hbm, dstb_hbm, src2d_hbm, dst2d_hbm, out_hbm,
             *rest):
        if with_deg:
            (deg_hbm, acc, isrc, idst, buf0, buf1, tsrc, tdst, sem0, sem1,
             dacc, dzbuf, ones) = rest
        else:
            acc, isrc, idst, buf0, buf1, tsrc, tdst, sem0, sem1 = rest
        c = lax.axis_index("c")
        s = lax.axis_index("s")
        wid = s * _NC + c
        zero16 = jnp.zeros((16,), _F32)

        @pl.loop(0, _BS)
        def _(i):
            @pl.loop(0, d, step=16)
            def _(j):
                buf0[i, pl.ds(j, 16)] = zero16

        off = 0
        while off < rps:
            step = min(_BS, rps - off)
            pltpu.sync_copy(buf0.at[pl.ds(0, step)],
                            acc.at[pl.ds(s * rps + off, step)])
            off += step
        if with_deg:
            one16 = jnp.ones((16,), _F32)

            @pl.loop(0, dzr)
            def _(i):
                dzbuf[i] = zero16

            @pl.loop(0, _BS)
            def _(i):
                ones[i] = one16

            for kk in range(4):
                pltpu.sync_copy(dzbuf,
                                dacc.at[pl.ds(s * rps + kk * dzr, dzr)])

        pltpu.sync_copy(srcb_hbm.at[pl.ds(wid * nblk, nblk)], isrc)
        pltpu.sync_copy(dstb_hbm.at[pl.ds(wid * nblk, nblk)], idst)
        plsc.subcore_barrier()

        def gather(blk, buf, sem):
            pltpu.async_copy(y_hbm.at[isrc.at[blk]], buf, sem)

        def gather_wait(blk, buf, sem):
            pltpu.make_async_copy(
                y_hbm.at[isrc.at[blk]], buf, sem).wait()

        def scatter(blk, buf):
            pltpu.sync_copy(buf, acc.at[idst.at[blk]], add=True)
            if with_deg:
                pltpu.sync_copy(ones, dacc.at[idst.at[blk]],
                                add=True)

        gather(0, buf0, sem0)
        npair = ((nblk + 1) // 2) * 2

        @pl.loop(0, npair, step=2)
        def _(t):
            gather_wait(t, buf0, sem0)

            @pl.when(t + 1 < nblk)
            def _():
                gather(t + 1, buf1, sem1)

            scatter(t, buf0)

            @pl.when(t + 1 < nblk)
            def _():
                gather_wait(t + 1, buf1, sem1)

                @pl.when(t + 2 < nblk)
                def _():
                    gather(t + 2, buf0, sem0)

                scatter(t + 1, buf1)

        if tail:
            @pl.when(wid < tail)
            def _():
                r = trow0 + wid
                pltpu.sync_copy(src2d_hbm.at[r], tsrc)
                pltpu.sync_copy(dst2d_hbm.at[r], tdst)
                pltpu.async_copy(y_hbm.at[tsrc],
                                 buf0.at[pl.ds(0, _K)], sem0).wait()
                pltpu.sync_copy(buf0.at[pl.ds(0, _K)],
                                acc.at[tdst], add=True)
                if with_deg:
                    pltpu.sync_copy(ones.at[pl.ds(0, _K)],
                                    dacc.at[tdst], add=True)

        plsc.subcore_barrier()
        pltpu.sync_copy(acc.at[pl.ds(s * rps, rps)],
                        out_hbm.at[c].at[pl.ds(s * rps, rps)])
        if with_deg:
            pltpu.sync_copy(dacc.at[pl.ds(s * rps, rps)],
                            deg_hbm.at[c].at[pl.ds(s * rps, rps)])

    f = pl.kernel(
        body, out_type=out_type, mesh=mesh, scratch_types=scratch,
        compiler_params=pltpu.CompilerParams(use_tc_tiling_on_sc=False))
    res = f(y, srcb, dstb, src2d, dst2d)
    return tuple(res) if with_deg else res[0]


def kernel(x, edge_index, W1l, b1, W1r, W2l, b2, W2r):
    e = edge_index.shape[1]
    src, dst = edge_index[0], edge_index[1]
    src2d = src.reshape(e // _K, _K)
    dst2d = dst.reshape(e // _K, _K)
    covered = (e // (_BS * _NW)) * _NW * _BS
    srcb = src[:covered].reshape(-1, _BS)
    dstb = dst[:covered].reshape(-1, _BS)

    y1 = _tc_matmul(x, W1l)
    r1 = _tc_matmul(x, W1r)
    aggp, degp = _sc_segment_sum(y1, srcb, dstb, src2d, dst2d,
                                 with_deg=True)
    h, y2 = _tc_layer1_post(aggp, degp, r1, b1, W2l)
    r2 = _tc_matmul(h, W2r)
    agg2p = _sc_segment_sum(y2, srcb, dstb, src2d, dst2d, with_deg=False)
    return _tc_layer2_post(agg2p, degp, r2, b2)
